# baseline (speedup 1.0000x reference)
import jax
import jax.numpy as jnp
from jax import lax
from jax.experimental import pallas as pl
from jax.experimental.pallas import tpu as pltpu
from jax.experimental.pallas import tpu_sc as plsc

DIM = 512
CODE_SIZE = 16384
CODE_DEPTH = 8
MAX_LEN = 2048
B = 16
T = MAX_LEN - 1
V = CODE_SIZE + 2
HALF = MAX_LEN // 2
CH = 32
NCH = HALF // CH
TOKROW = 16384


def _body(tokens_hbm, lengths_hbm, table_hbm, eot_hbm, eot2_hbm,
          out1_hbm, out2_hbm, mask_hbm,
          tokbuf, idxbuf, acc, r0, r1, r2, r3, lenbuf, e1buf, e2buf, maskbuf,
          sem, sema, sem0, sem1, sem2, sem3):
    cid = lax.axis_index("c")
    sid = lax.axis_index("s")
    wid = sid * 2 + cid
    b = wid // 2
    h = wid % 2
    row0 = h * HALF
    lanes = lax.iota(jnp.int32, 16)

    pltpu.sync_copy(tokens_hbm.at[b], tokbuf)
    pltpu.sync_copy(eot_hbm, e1buf)
    pltpu.sync_copy(eot2_hbm, e2buf)

    lenbuf[0, pl.ds(0, 16)] = jnp.full((16,), b, jnp.int32)
    pltpu.async_copy(lengths_hbm.at[lenbuf.at[0]], lenbuf.at[1], sem).wait()
    len2v = jnp.minimum(lenbuf[1, pl.ds(0, 16)] + jnp.full((16,), 1, jnp.int32),
                        jnp.full((16,), MAX_LEN, jnp.int32))

    def mask_body(j, carry):
        pos = h * HALF + j * 16
        posv = lanes + jnp.full((16,), pos, jnp.int32)
        diff = len2v - posv
        zero = jnp.full((16,), 0, jnp.int32)
        one = jnp.full((16,), 1, jnp.int32)
        maskbuf[pl.ds(j * 16, 16)] = jnp.minimum(jnp.maximum(diff, zero), one)
        return carry

    lax.fori_loop(0, HALF // 16, mask_body, 0)
    pltpu.sync_copy(maskbuf, mask_hbm.at[b].at[pl.ds(row0, HALF)])

    def chunk_body(c, carry):
        pos0 = c * CH
        for k in range(CODE_DEPTH):
            for j in range(CH // 16):
                off = k * T + h * HALF + pos0 + j * 16
                idxbuf[k, pl.ds(j * 16, 16)] = tokbuf[pl.ds(off, 16)]

        first_sc = (1 - h) * (1 - jnp.minimum(c, 1))
        indf = jnp.full((16,), first_sc.astype(jnp.float32), jnp.float32)

        bufs = (r0, r1, r2, r3)
        sems = (sem0, sem1, sem2, sem3)

        def fire(k):
            i = (k - 1) % 4
            return pltpu.async_copy(
                table_hbm.at[k].at[idxbuf.at[k]], bufs[i], sems[i])

        cp_t = pltpu.async_copy(table_hbm.at[0].at[idxbuf.at[0]], r0, sem0)
        cp_a = pltpu.async_copy(table_hbm.at[1].at[idxbuf.at[1]], acc, sema)
        cps = {2: fire(2), 3: fire(3), 4: fire(4)}

        cp_t.wait()
        for q in range(DIM // 16):
            sl = pl.ds(q * 16, 16)
            t0v = r0[0, sl]
            r0[0, sl] = t0v + indf * (e1buf[sl] - t0v)
        pltpu.sync_copy(r0, out1_hbm.at[b].at[pl.ds(row0 + pos0, CH)])
        cps[5] = fire(5)

        cp_a.wait()
        for k in range(2, CODE_DEPTH):
            cps[k].wait()
            t = bufs[(k - 1) % 4]

            def add_row(r, inner):
                a = acc.at[r]
                tt = t.at[r]
                for q in range(DIM // 16):
                    sl = pl.ds(q * 16, 16)
                    plsc.addupdate(a.at[pl.ds(q * 16, 16)], tt[sl])
                return inner

            lax.fori_loop(0, CH, add_row, 0)
            if k + 4 < CODE_DEPTH:
                cps[k + 4] = fire(k + 4)
        for q in range(DIM // 16):
            sl = pl.ds(q * 16, 16)
            a0 = acc[0, sl]
            acc[0, sl] = a0 + indf * (e2buf[sl] - a0)
        pltpu.sync_copy(acc, out2_hbm.at[b].at[pl.ds(row0 + pos0, CH)])
        return carry

    lax.fori_loop(0, NCH, chunk_body, 0)


def kernel(tokens, lengths, emb, EOT_emb, layer2_EOT_emb):
    tokens_p = jnp.pad(tokens, ((0, 0), (1, TOKROW - CODE_DEPTH * T - 1)))
    mesh = plsc.VectorSubcoreMesh(core_axis_name="c", subcore_axis_name="s")
    out1, out2, mask = pl.kernel(
        _body,
        out_type=(
            jax.ShapeDtypeStruct((B, MAX_LEN, DIM), jnp.float32),
            jax.ShapeDtypeStruct((B, MAX_LEN, DIM), jnp.float32),
            jax.ShapeDtypeStruct((B, MAX_LEN), jnp.int32),
        ),
        mesh=mesh,
        scratch_types=[
            pltpu.VMEM((TOKROW,), jnp.int32),
            pltpu.VMEM((CODE_DEPTH, CH), jnp.int32),
            pltpu.VMEM((CH, DIM), jnp.float32),
            pltpu.VMEM((CH, DIM), jnp.float32),
            pltpu.VMEM((CH, DIM), jnp.float32),
            pltpu.VMEM((CH, DIM), jnp.float32),
            pltpu.VMEM((CH, DIM), jnp.float32),
            pltpu.VMEM((2, 16), jnp.int32),
            pltpu.VMEM((DIM,), jnp.float32),
            pltpu.VMEM((DIM,), jnp.float32),
            pltpu.VMEM((HALF,), jnp.int32),
            pltpu.SemaphoreType.DMA,
            pltpu.SemaphoreType.DMA,
            pltpu.SemaphoreType.DMA,
            pltpu.SemaphoreType.DMA,
            pltpu.SemaphoreType.DMA,
            pltpu.SemaphoreType.DMA,
        ],
    )(tokens_p, lengths, emb, EOT_emb.reshape(DIM), layer2_EOT_emb.reshape(DIM))
    return (out1, out2, mask)


# speedup vs baseline: 1.6619x; 1.6619x over previous
import jax
import jax.numpy as jnp
from jax import lax
from jax.experimental import pallas as pl
from jax.experimental.pallas import tpu as pltpu
from jax.experimental.pallas import tpu_sc as plsc

DIM = 512
CODE_SIZE = 16384
CODE_DEPTH = 8
MAX_LEN = 2048
B = 16
T = MAX_LEN - 1
V = CODE_SIZE + 2
HALF = MAX_LEN // 2
CH = 64
NCH = HALF // CH
TOKROW = 16384


def _body(tokens_hbm, lengths_hbm, table_hbm, eot_hbm, eot2_hbm,
          out1_hbm, out2_hbm, mask_hbm,
          tokbuf, idxbuf, acc, b0, t0, lenbuf, e1buf, e2buf, maskbuf,
          sem, semt, sema, semb0):
    cid = lax.axis_index("c")
    sid = lax.axis_index("s")
    wid = sid * 2 + cid
    b = wid // 2
    h = wid % 2
    row0 = h * HALF
    lanes = lax.iota(jnp.int32, 16)

    pltpu.sync_copy(tokens_hbm.at[b], tokbuf)
    pltpu.sync_copy(eot_hbm, e1buf)
    pltpu.sync_copy(eot2_hbm, e2buf)

    lenbuf[0, pl.ds(0, 16)] = jnp.full((16,), b, jnp.int32)
    pltpu.async_copy(lengths_hbm.at[lenbuf.at[0]], lenbuf.at[1], sem).wait()
    len2v = jnp.minimum(lenbuf[1, pl.ds(0, 16)] + jnp.full((16,), 1, jnp.int32),
                        jnp.full((16,), MAX_LEN, jnp.int32))

    def mask_body(j, carry):
        pos = h * HALF + j * 16
        posv = lanes + jnp.full((16,), pos, jnp.int32)
        diff = len2v - posv
        zero = jnp.full((16,), 0, jnp.int32)
        one = jnp.full((16,), 1, jnp.int32)
        maskbuf[pl.ds(j * 16, 16)] = jnp.minimum(jnp.maximum(diff, zero), one)
        return carry

    lax.fori_loop(0, HALF // 16, mask_body, 0)
    pltpu.sync_copy(maskbuf, mask_hbm.at[b].at[pl.ds(row0, HALF)])

    def chunk_body(c, carry):
        pos0 = c * CH
        for k in range(CODE_DEPTH):
            for j in range(CH // 16):
                off = k * T + h * HALF + pos0 + j * 16
                idxbuf[k, pl.ds(j * 16, 16)] = tokbuf[pl.ds(off, 16)]

        first_sc = (1 - h) * (1 - jnp.minimum(c, 1))
        indf = jnp.full((16,), first_sc.astype(jnp.float32), jnp.float32)

        cp_t = pltpu.async_copy(table_hbm.at[0].at[idxbuf.at[0]], t0, semt)
        cp_a = pltpu.async_copy(table_hbm.at[1].at[idxbuf.at[1]], acc, sema)
        bufs = (b0, t0)
        sems = (semb0, semt)
        cps = [pltpu.async_copy(table_hbm.at[2].at[idxbuf.at[2]], b0, semb0), None]

        cp_t.wait()
        for q in range(DIM // 16):
            sl = pl.ds(q * 16, 16)
            t0v = t0[0, sl]
            t0[0, sl] = t0v + indf * (e1buf[sl] - t0v)
        pltpu.sync_copy(t0, out1_hbm.at[b].at[pl.ds(row0 + pos0, CH)])
        cps[1] = pltpu.async_copy(table_hbm.at[3].at[idxbuf.at[3]], t0, semt)

        cp_a.wait()
        for k in range(2, CODE_DEPTH):
            cur = k % 2
            cps[cur].wait()
            t = bufs[cur]

            def add_row(r, inner):
                a = acc.at[r]
                tt = t.at[r]
                for q in range(DIM // 16):
                    sl = pl.ds(q * 16, 16)
                    plsc.addupdate(a.at[pl.ds(q * 16, 16)], tt[sl])
                return inner

            lax.fori_loop(0, CH, add_row, 0)
            if k + 2 < CODE_DEPTH:
                cps[cur] = pltpu.async_copy(
                    table_hbm.at[k + 2].at[idxbuf.at[k + 2]], bufs[cur],
                    sems[cur])
        for q in range(DIM // 16):
            sl = pl.ds(q * 16, 16)
            a0 = acc[0, sl]
            acc[0, sl] = a0 + indf * (e2buf[sl] - a0)
        pltpu.sync_copy(acc, out2_hbm.at[b].at[pl.ds(row0 + pos0, CH)])
        return carry

    lax.fori_loop(0, NCH, chunk_body, 0)


def kernel(tokens, lengths, emb, EOT_emb, layer2_EOT_emb):
    tokens_p = jnp.pad(tokens, ((0, 0), (1, TOKROW - CODE_DEPTH * T - 1)))
    mesh = plsc.VectorSubcoreMesh(core_axis_name="c", subcore_axis_name="s")
    out1, out2, mask = pl.kernel(
        _body,
        out_type=(
            jax.ShapeDtypeStruct((B, MAX_LEN, DIM), jnp.float32),
            jax.ShapeDtypeStruct((B, MAX_LEN, DIM), jnp.float32),
            jax.ShapeDtypeStruct((B, MAX_LEN), jnp.int32),
        ),
        mesh=mesh,
        compiler_params=pltpu.CompilerParams(use_tc_tiling_on_sc=True),
        scratch_types=[
            pltpu.VMEM((TOKROW,), jnp.int32),
            pltpu.VMEM((CODE_DEPTH, CH), jnp.int32),
            pltpu.VMEM((CH, DIM), jnp.float32),
            pltpu.VMEM((CH, DIM), jnp.float32),
            pltpu.VMEM((CH, DIM), jnp.float32),
            pltpu.VMEM((2, 16), jnp.int32),
            pltpu.VMEM((DIM,), jnp.float32),
            pltpu.VMEM((DIM,), jnp.float32),
            pltpu.VMEM((HALF,), jnp.int32),
            pltpu.SemaphoreType.DMA,
            pltpu.SemaphoreType.DMA,
            pltpu.SemaphoreType.DMA,
            pltpu.SemaphoreType.DMA,
        ],
    )(tokens_p, lengths, emb, EOT_emb.reshape(DIM), layer2_EOT_emb.reshape(DIM))
    return (out1, out2, mask)


# speedup vs baseline: 1.6645x; 1.0015x over previous
import jax
import jax.numpy as jnp
from jax import lax
from jax.experimental import pallas as pl
from jax.experimental.pallas import tpu as pltpu
from jax.experimental.pallas import tpu_sc as plsc
from jax.experimental import layout as jex_layout

DIM = 512
CODE_SIZE = 16384
CODE_DEPTH = 8
MAX_LEN = 2048
B = 16
T = MAX_LEN - 1
V = CODE_SIZE + 2
HALF = MAX_LEN // 2
CH = 64
NCH = HALF // CH
TOKROW = 16384


def _body(tokens_hbm, lengths_hbm, table_hbm, eot_hbm, eot2_hbm,
          out1_hbm, out2_hbm, mask_hbm,
          tokbuf, idxbuf, acc, b0, t0, lenbuf, e1buf, e2buf, maskbuf,
          sem, semt, sema, semb0):
    cid = lax.axis_index("c")
    sid = lax.axis_index("s")
    wid = sid * 2 + cid
    b = wid // 2
    h = wid % 2
    row0 = h * HALF
    lanes = lax.iota(jnp.int32, 16)

    pltpu.sync_copy(tokens_hbm.at[b], tokbuf)
    pltpu.sync_copy(eot_hbm, e1buf)
    pltpu.sync_copy(eot2_hbm, e2buf)

    lenbuf[0, pl.ds(0, 16)] = jnp.full((16,), b, jnp.int32)
    pltpu.async_copy(lengths_hbm.at[lenbuf.at[0]], lenbuf.at[1], sem).wait()
    len2v = jnp.minimum(lenbuf[1, pl.ds(0, 16)] + jnp.full((16,), 1, jnp.int32),
                        jnp.full((16,), MAX_LEN, jnp.int32))

    def mask_body(j, carry):
        pos = h * HALF + j * 16
        posv = lanes + jnp.full((16,), pos, jnp.int32)
        diff = len2v - posv
        zero = jnp.full((16,), 0, jnp.int32)
        one = jnp.full((16,), 1, jnp.int32)
        maskbuf[pl.ds(j * 16, 16)] = jnp.minimum(jnp.maximum(diff, zero), one)
        return carry

    lax.fori_loop(0, HALF // 16, mask_body, 0)
    pltpu.sync_copy(maskbuf, mask_hbm.at[b].at[pl.ds(row0, HALF)])

    def chunk_body(c, carry):
        pos0 = c * CH
        for k in range(CODE_DEPTH):
            for j in range(CH // 16):
                off = k * T + h * HALF + pos0 + j * 16
                idxbuf[k, pl.ds(j * 16, 16)] = tokbuf[pl.ds(off, 16)]

        first_sc = (1 - h) * (1 - jnp.minimum(c, 1))
        indf = jnp.full((16,), first_sc.astype(jnp.float32), jnp.float32)

        cp_t = pltpu.async_copy(table_hbm.at[0].at[idxbuf.at[0]], t0, semt)
        cp_a = pltpu.async_copy(table_hbm.at[1].at[idxbuf.at[1]], acc, sema)
        bufs = (b0, t0)
        sems = (semb0, semt)
        cps = [pltpu.async_copy(table_hbm.at[2].at[idxbuf.at[2]], b0, semb0), None]

        cp_t.wait()
        for q in range(DIM // 16):
            sl = pl.ds(q * 16, 16)
            t0v = t0[0, sl]
            t0[0, sl] = t0v + indf * (e1buf[sl] - t0v)
        pltpu.sync_copy(t0, out1_hbm.at[b].at[pl.ds(row0 + pos0, CH)])
        cps[1] = pltpu.async_copy(table_hbm.at[3].at[idxbuf.at[3]], t0, semt)

        cp_a.wait()
        for k in range(2, CODE_DEPTH):
            cur = k % 2
            cps[cur].wait()
            t = bufs[cur]

            def add_row(r, inner):
                a = acc.at[r]
                tt = t.at[r]
                for q in range(DIM // 16):
                    sl = pl.ds(q * 16, 16)
                    plsc.addupdate(a.at[pl.ds(q * 16, 16)], tt[sl])
                return inner

            lax.fori_loop(0, CH, add_row, 0)
            if k + 2 < CODE_DEPTH:
                cps[cur] = pltpu.async_copy(
                    table_hbm.at[k + 2].at[idxbuf.at[k + 2]], bufs[cur],
                    sems[cur])
        for q in range(DIM // 16):
            sl = pl.ds(q * 16, 16)
            a0 = acc[0, sl]
            acc[0, sl] = a0 + indf * (e2buf[sl] - a0)
        pltpu.sync_copy(acc, out2_hbm.at[b].at[pl.ds(row0 + pos0, CH)])
        return carry

    lax.fori_loop(0, NCH, chunk_body, 0)


def _kernel_impl(tokens, lengths, emb, EOT_emb, layer2_EOT_emb):
    tokens_p = jnp.pad(tokens, ((0, 0), (1, TOKROW - CODE_DEPTH * T - 1)))
    mesh = plsc.VectorSubcoreMesh(core_axis_name="c", subcore_axis_name="s")
    out1, out2, mask = pl.kernel(
        _body,
        out_type=(
            jax.ShapeDtypeStruct((B, MAX_LEN, DIM), jnp.float32),
            jax.ShapeDtypeStruct((B, MAX_LEN, DIM), jnp.float32),
            jax.ShapeDtypeStruct((B, MAX_LEN), jnp.int32),
        ),
        mesh=mesh,
        compiler_params=pltpu.CompilerParams(use_tc_tiling_on_sc=True),
        scratch_types=[
            pltpu.VMEM((TOKROW,), jnp.int32),
            pltpu.VMEM((CODE_DEPTH, CH), jnp.int32),
            pltpu.VMEM((CH, DIM), jnp.float32),
            pltpu.VMEM((CH, DIM), jnp.float32),
            pltpu.VMEM((CH, DIM), jnp.float32),
            pltpu.VMEM((2, 16), jnp.int32),
            pltpu.VMEM((DIM,), jnp.float32),
            pltpu.VMEM((DIM,), jnp.float32),
            pltpu.VMEM((HALF,), jnp.int32),
            pltpu.SemaphoreType.DMA,
            pltpu.SemaphoreType.DMA,
            pltpu.SemaphoreType.DMA,
            pltpu.SemaphoreType.DMA,
        ],
    )(tokens_p, lengths, emb, EOT_emb.reshape(DIM), layer2_EOT_emb.reshape(DIM))
    return (out1, out2, mask)


_inner = None


def kernel(tokens, lengths, emb, EOT_emb, layer2_EOT_emb):
    global _inner
    if _inner is None:
        shd = jax.sharding.SingleDeviceSharding(jax.devices()[0])
        fmt = jex_layout.Format(
            jex_layout.Layout(major_to_minor=(0, 1, 2)), shd)
        _inner = jax.jit(_kernel_impl,
                         in_shardings=(shd, shd, fmt, shd, shd))
    return _inner(tokens, lengths, emb, EOT_emb, layer2_EOT_emb)


# speedup vs baseline: 2.3566x; 1.4158x over previous
import jax
import jax.numpy as jnp
from jax import lax
from jax.experimental import pallas as pl
from jax.experimental.pallas import tpu as pltpu
from jax.experimental.pallas import tpu_sc as plsc
from jax.experimental import layout as jex_layout

DIM = 512
CODE_SIZE = 16384
CODE_DEPTH = 8
MAX_LEN = 2048
B = 16
T = MAX_LEN - 1
V = CODE_SIZE + 2
HALF = MAX_LEN // 2
CH = 64
NCH = HALF // CH
TOKROW = 16384


def _body(tokens_hbm, lengths_hbm, table_hbm, eot_hbm, eot2_hbm,
          out1_hbm, out2_hbm, mask_hbm,
          tokbuf, idxbuf, acc, b0, t0, lenbuf, e1buf, e2buf, maskbuf,
          sem, semt, sema, semb0):
    cid = lax.axis_index("c")
    sid = lax.axis_index("s")
    wid = sid * 2 + cid
    b = wid // 2
    h = wid % 2
    row0 = h * HALF
    lanes = lax.iota(jnp.int32, 16)

    pltpu.sync_copy(tokens_hbm.at[b], tokbuf)
    pltpu.sync_copy(eot_hbm, e1buf)
    pltpu.sync_copy(eot2_hbm, e2buf)

    lenbuf[0, pl.ds(0, 16)] = jnp.full((16,), b, jnp.int32)
    pltpu.async_copy(lengths_hbm.at[lenbuf.at[0]], lenbuf.at[1], sem).wait()
    len2v = jnp.minimum(lenbuf[1, pl.ds(0, 16)] + jnp.full((16,), 1, jnp.int32),
                        jnp.full((16,), MAX_LEN, jnp.int32))

    def mask_body(j, carry):
        pos = h * HALF + j * 16
        posv = lanes + jnp.full((16,), pos, jnp.int32)
        diff = len2v - posv
        zero = jnp.full((16,), 0, jnp.int32)
        one = jnp.full((16,), 1, jnp.int32)
        maskbuf[pl.ds(j * 16, 16)] = jnp.minimum(jnp.maximum(diff, zero), one)
        return carry

    lax.fori_loop(0, HALF // 16, mask_body, 0)
    pltpu.sync_copy(maskbuf, mask_hbm.at[b].at[pl.ds(row0, HALF)])

    def chunk_body(c, carry):
        pos0 = c * CH
        for k in range(CODE_DEPTH):
            for j in range(CH // 16):
                off = k * T + h * HALF + pos0 + j * 16
                idxbuf[k, pl.ds(j * 16, 16)] = (
                    tokbuf[pl.ds(off, 16)] * jnp.full((16,), CODE_DEPTH, jnp.int32)
                    + jnp.full((16,), k, jnp.int32))

        first_sc = (1 - h) * (1 - jnp.minimum(c, 1))
        indf = jnp.full((16,), first_sc.astype(jnp.float32), jnp.float32)

        cp_t = pltpu.async_copy(table_hbm.at[idxbuf.at[0]], t0, semt)
        cp_a = pltpu.async_copy(table_hbm.at[idxbuf.at[1]], acc, sema)
        bufs = (b0, t0)
        sems = (semb0, semt)
        cps = [pltpu.async_copy(table_hbm.at[idxbuf.at[2]], b0, semb0), None]

        cp_t.wait()
        for q in range(DIM // 16):
            sl = pl.ds(q * 16, 16)
            t0v = t0[0, sl]
            t0[0, sl] = t0v + indf * (e1buf[sl] - t0v)
        pltpu.sync_copy(t0, out1_hbm.at[b].at[pl.ds(row0 + pos0, CH)])
        cps[1] = pltpu.async_copy(table_hbm.at[idxbuf.at[3]], t0, semt)

        cp_a.wait()
        for k in range(2, CODE_DEPTH):
            cur = k % 2
            cps[cur].wait()
            t = bufs[cur]

            def add_row(r, inner):
                a = acc.at[r]
                tt = t.at[r]
                for q in range(DIM // 16):
                    sl = pl.ds(q * 16, 16)
                    plsc.addupdate(a.at[pl.ds(q * 16, 16)], tt[sl])
                return inner

            lax.fori_loop(0, CH, add_row, 0)
            if k + 2 < CODE_DEPTH:
                cps[cur] = pltpu.async_copy(
                    table_hbm.at[idxbuf.at[k + 2]], bufs[cur], sems[cur])
        for q in range(DIM // 16):
            sl = pl.ds(q * 16, 16)
            a0 = acc[0, sl]
            acc[0, sl] = a0 + indf * (e2buf[sl] - a0)
        pltpu.sync_copy(acc, out2_hbm.at[b].at[pl.ds(row0 + pos0, CH)])
        return carry

    lax.fori_loop(0, NCH, chunk_body, 0)


def _kernel_impl(tokens, lengths, emb, EOT_emb, layer2_EOT_emb):
    tokens_p = jnp.pad(tokens, ((0, 0), (1, TOKROW - CODE_DEPTH * T - 1)))
    table = jnp.transpose(emb, (1, 0, 2)).reshape(V * CODE_DEPTH, DIM)
    mesh = plsc.VectorSubcoreMesh(core_axis_name="c", subcore_axis_name="s")
    out1, out2, mask = pl.kernel(
        _body,
        out_type=(
            jax.ShapeDtypeStruct((B, MAX_LEN, DIM), jnp.float32),
            jax.ShapeDtypeStruct((B, MAX_LEN, DIM), jnp.float32),
            jax.ShapeDtypeStruct((B, MAX_LEN), jnp.int32),
        ),
        mesh=mesh,
        compiler_params=pltpu.CompilerParams(use_tc_tiling_on_sc=True),
        scratch_types=[
            pltpu.VMEM((TOKROW,), jnp.int32),
            pltpu.VMEM((CODE_DEPTH, CH), jnp.int32),
            pltpu.VMEM((CH, DIM), jnp.float32),
            pltpu.VMEM((CH, DIM), jnp.float32),
            pltpu.VMEM((CH, DIM), jnp.float32),
            pltpu.VMEM((2, 16), jnp.int32),
            pltpu.VMEM((DIM,), jnp.float32),
            pltpu.VMEM((DIM,), jnp.float32),
            pltpu.VMEM((HALF,), jnp.int32),
            pltpu.SemaphoreType.DMA,
            pltpu.SemaphoreType.DMA,
            pltpu.SemaphoreType.DMA,
            pltpu.SemaphoreType.DMA,
        ],
    )(tokens_p, lengths, table, EOT_emb.reshape(DIM), layer2_EOT_emb.reshape(DIM))
    return (out1, out2, mask)


kernel = _kernel_impl


# speedup vs baseline: 2.3620x; 1.0023x over previous
import jax
import jax.numpy as jnp
from jax import lax
from jax.experimental import pallas as pl
from jax.experimental.pallas import tpu as pltpu
from jax.experimental.pallas import tpu_sc as plsc

DIM = 512
CODE_SIZE = 16384
CODE_DEPTH = 8
MAX_LEN = 2048
B = 16
T = MAX_LEN - 1
V = CODE_SIZE + 2
HALF = MAX_LEN // 2
CH = 64
NCH = HALF // CH
TOKROW = 16384


def _body(tokens_hbm, lengths_hbm, table_hbm, eot_hbm, eot2_hbm,
          out1_hbm, out2_hbm, mask_hbm,
          tokbuf, idxbuf, acc, b0, t0, lenbuf, e1buf, e2buf, maskbuf,
          sem, semt, sema, semb0):
    cid = lax.axis_index("c")
    sid = lax.axis_index("s")
    wid = sid * 2 + cid
    b = wid // 2
    h = wid % 2
    row0 = h * HALF
    lanes = lax.iota(jnp.int32, 16)

    pltpu.sync_copy(tokens_hbm.at[b], tokbuf)
    pltpu.sync_copy(eot_hbm, e1buf)
    pltpu.sync_copy(eot2_hbm, e2buf)

    lenbuf[0, pl.ds(0, 16)] = jnp.full((16,), b, jnp.int32)
    pltpu.async_copy(lengths_hbm.at[lenbuf.at[0]], lenbuf.at[1], sem).wait()
    len2v = jnp.minimum(lenbuf[1, pl.ds(0, 16)] + jnp.full((16,), 1, jnp.int32),
                        jnp.full((16,), MAX_LEN, jnp.int32))

    def mask_body(j, carry):
        pos = h * HALF + j * 16
        posv = lanes + jnp.full((16,), pos, jnp.int32)
        diff = len2v - posv
        zero = jnp.full((16,), 0, jnp.int32)
        one = jnp.full((16,), 1, jnp.int32)
        maskbuf[pl.ds(j * 16, 16)] = jnp.minimum(jnp.maximum(diff, zero), one)
        return carry

    lax.fori_loop(0, HALF // 16, mask_body, 0)
    pltpu.sync_copy(maskbuf, mask_hbm.at[b].at[pl.ds(row0, HALF)])

    def chunk_body(c, carry):
        pos0 = c * CH
        for k in range(CODE_DEPTH):
            for j in range(CH // 16):
                off = k * T + h * HALF + pos0 + j * 16
                idxbuf[k, pl.ds(j * 16, 16)] = (
                    tokbuf[pl.ds(off, 16)] * jnp.full((16,), CODE_DEPTH, jnp.int32)
                    + jnp.full((16,), k, jnp.int32))

        first_sc = (1 - h) * (1 - jnp.minimum(c, 1))
        indf = jnp.full((16,), first_sc.astype(jnp.float32), jnp.float32)

        cp_t = pltpu.async_copy(table_hbm.at[idxbuf.at[0]], t0, semt)
        cp_a = pltpu.async_copy(table_hbm.at[idxbuf.at[1]], acc, sema)
        bufs = (b0, t0)
        sems = (semb0, semt)
        cps = [pltpu.async_copy(table_hbm.at[idxbuf.at[2]], b0, semb0), None]

        cp_t.wait()
        for q in range(DIM // 16):
            sl = pl.ds(q * 16, 16)
            t0v = t0[0, sl]
            t0[0, sl] = t0v + indf * (e1buf[sl] - t0v)
        pltpu.sync_copy(t0, out1_hbm.at[b].at[pl.ds(row0 + pos0, CH)])
        cps[1] = pltpu.async_copy(table_hbm.at[idxbuf.at[3]], t0, semt)

        cp_a.wait()
        for k in range(2, CODE_DEPTH):
            cur = k % 2
            cps[cur].wait()
            t = bufs[cur]

            def add_row(r, inner):
                a = acc.at[r]
                tt = t.at[r]
                for q in range(DIM // 16):
                    sl = pl.ds(q * 16, 16)
                    plsc.addupdate(a.at[pl.ds(q * 16, 16)], tt[sl])
                return inner

            lax.fori_loop(0, CH, add_row, 0)
            if k + 2 < CODE_DEPTH:
                cps[cur] = pltpu.async_copy(
                    table_hbm.at[idxbuf.at[k + 2]], bufs[cur], sems[cur])
        for q in range(DIM // 16):
            sl = pl.ds(q * 16, 16)
            a0 = acc[0, sl]
            acc[0, sl] = a0 + indf * (e2buf[sl] - a0)
        pltpu.sync_copy(acc, out2_hbm.at[b].at[pl.ds(row0 + pos0, CH)])
        return carry

    lax.fori_loop(0, NCH, chunk_body, 0)


def kernel(tokens, lengths, emb, EOT_emb, layer2_EOT_emb):
    tokens_p = jnp.pad(tokens, ((0, 0), (1, TOKROW - CODE_DEPTH * T - 1)))
    table = jnp.transpose(emb, (1, 0, 2)).reshape(V * CODE_DEPTH, DIM)
    mesh = plsc.VectorSubcoreMesh(core_axis_name="c", subcore_axis_name="s")
    out1, out2, mask = pl.kernel(
        _body,
        out_type=(
            jax.ShapeDtypeStruct((B, MAX_LEN, DIM), jnp.float32),
            jax.ShapeDtypeStruct((B, MAX_LEN, DIM), jnp.float32),
            jax.ShapeDtypeStruct((B, MAX_LEN), jnp.int32),
        ),
        mesh=mesh,
        compiler_params=pltpu.CompilerParams(use_tc_tiling_on_sc=True),
        scratch_types=[
            pltpu.VMEM((TOKROW,), jnp.int32),
            pltpu.VMEM((CODE_DEPTH, CH), jnp.int32),
            pltpu.VMEM((CH, DIM), jnp.float32),
            pltpu.VMEM((CH, DIM), jnp.float32),
            pltpu.VMEM((CH, DIM), jnp.float32),
            pltpu.VMEM((2, 16), jnp.int32),
            pltpu.VMEM((DIM,), jnp.float32),
            pltpu.VMEM((DIM,), jnp.float32),
            pltpu.VMEM((HALF,), jnp.int32),
            pltpu.SemaphoreType.DMA,
            pltpu.SemaphoreType.DMA,
            pltpu.SemaphoreType.DMA,
            pltpu.SemaphoreType.DMA,
        ],
    )(tokens_p, lengths, table, EOT_emb.reshape(DIM), layer2_EOT_emb.reshape(DIM))
    return (out1, out2, mask)
